# baseline (speedup 1.0000x reference)
import functools

import jax
import jax.numpy as jnp
from jax import lax
from jax.experimental import pallas as pl
from jax.experimental.pallas import tpu as pltpu
from jax.experimental.pallas import tpu_sc as plsc

_D = 1024
_S = 16384
_K = 16
_MINK = 4
_T = 2048
_BT = 128
_NSTEPS = _T // _BT
_NW = 32
_TPW = _T // _NW
_LC = 16


_BSR = 2048
_NJ = _S // _BSR


def _router_body(x_ref, w_ref, idx_ref, wts_ref, psum_ref, cnt_ref, lg_ref):
    step = pl.program_id(0)
    j = pl.program_id(1)

    @pl.when((step == 0) & (j == 0))
    def _():
        psum_ref[...] = jnp.zeros_like(psum_ref)
        cnt_ref[...] = jnp.zeros_like(cnt_ref)

    lg_ref[:, pl.ds(j * _BSR, _BSR)] = jnp.dot(
        x_ref[...], w_ref[...], preferred_element_type=jnp.float32)

    @pl.when(j == _NJ - 1)
    def _finish():
        _router_finish(idx_ref, wts_ref, psum_ref, cnt_ref, lg_ref)


def _router_finish(idx_ref, wts_ref, psum_ref, cnt_ref, lg_ref):
    logits = lg_ref[...]

    m = jnp.max(logits, axis=1, keepdims=True)
    e = jnp.exp(logits - m)
    z = jnp.sum(e, axis=1, keepdims=True)
    psum_ref[...] += jnp.sum(e / z, axis=0, keepdims=True)

    iota_s = lax.broadcasted_iota(jnp.int32, logits.shape, 1)
    row = logits
    vals = []
    idxs = []
    neg = jnp.float32(-jnp.inf)
    cnt = jnp.zeros((1, _S), jnp.float32)
    for _ in range(_K):
        mk = jnp.max(row, axis=1, keepdims=True)
        ak = jnp.min(jnp.where(row == mk, iota_s, _S), axis=1, keepdims=True)
        vals.append(mk)
        idxs.append(ak)
        hit = iota_s == ak
        cnt = cnt + jnp.sum(hit.astype(jnp.float32), axis=0, keepdims=True)
        row = jnp.where(hit, neg, row)
    cnt_ref[...] += cnt
    tv = jnp.concatenate(vals, axis=1)
    ti = jnp.concatenate(idxs, axis=1)

    e2 = jnp.exp(tv - tv[:, :1])
    w = e2 / jnp.sum(e2, axis=1, keepdims=True)
    ranks = lax.broadcasted_iota(jnp.int32, w.shape, 1)
    keep = (ranks < _MINK) | (w > (1.0 / _K))
    w = w * keep.astype(jnp.float32)
    w = w / (jnp.sum(w, axis=1, keepdims=True) + 1e-9)

    idx_ref[...] = ti
    wts_ref[...] = w


@jax.jit
def _router(xf, w_router):
    return pl.pallas_call(
        _router_body,
        grid=(_NSTEPS, _NJ),
        in_specs=[
            pl.BlockSpec((_BT, _D), lambda i, j: (i, 0)),
            pl.BlockSpec((_D, _BSR), lambda i, j: (0, j)),
        ],
        out_specs=[
            pl.BlockSpec((_BT, _K), lambda i, j: (i, 0)),
            pl.BlockSpec((_BT, _K), lambda i, j: (i, 0)),
            pl.BlockSpec((1, _S), lambda i, j: (0, 0)),
            pl.BlockSpec((1, _S), lambda i, j: (0, 0)),
        ],
        out_shape=[
            jax.ShapeDtypeStruct((_T, _K), jnp.int32),
            jax.ShapeDtypeStruct((_T, _K), jnp.float32),
            jax.ShapeDtypeStruct((1, _S), jnp.float32),
            jax.ShapeDtypeStruct((1, _S), jnp.float32),
        ],
        scratch_shapes=[pltpu.VMEM((_BT, _S), jnp.float32)],
    )(xf, w_router)


def _moe_tile(x_hbm, pp_hbm, idx_hbm, wts_hbm, out_hbm,
              x_v, idx_v, w_v, rows_a, rows_b, out_a, out_b,
              sem_a, sem_b, sem_oa, sem_ob):
    wid = lax.axis_index("s") * 2 + lax.axis_index("c")
    base = wid * _TPW
    lanes = lax.iota(jnp.int32, _LC)
    zero16 = jnp.zeros((_LC,), jnp.float32)

    pltpu.sync_copy(x_hbm.at[pl.ds(base * _D, _TPW * _D)], x_v)
    pltpu.sync_copy(idx_hbm.at[pl.ds(base * _K, _TPW * _K)], idx_v)
    pltpu.sync_copy(wts_hbm.at[pl.ds(base * _K, _TPW * _K)], w_v)

    def _gather(t, rows, sem):
        return pltpu.make_async_copy(
            pp_hbm.at[idx_v.at[pl.ds(t * _K, _K)]], rows, sem)

    def _compute(t, rows, out_v):
        xoff = t * _D

        def _pbody(dc, accs):
            xv = x_v[pl.ds(xoff + dc * _LC, _LC)]
            return tuple(accs[k] + xv * rows[k, pl.ds(dc * _LC, _LC)]
                         for k in range(_K))

        accs = lax.fori_loop(0, _D // _LC, _pbody,
                             tuple(zero16 for _ in range(_K)))

        wv = w_v[pl.ds(t * _K, _K)]
        wps = []
        for k in range(_K):
            s_k = jnp.sum(accs[k], axis=0)
            w_k = jnp.sum(jnp.where(lanes == k, wv, 0.0), axis=0)
            wps.append(jnp.full((_LC,), s_k * w_k, jnp.float32))

        def _obody(dc, c2):
            oc = wps[0] * rows[0, pl.ds(dc * _LC, _LC)]
            for k in range(1, _K):
                oc = oc + wps[k] * rows[k, pl.ds(dc * _LC, _LC)]
            out_v[pl.ds(dc * _LC, _LC)] = oc
            return c2

        lax.fori_loop(0, _D // _LC, _obody, 0)

    _gather(0, rows_a, sem_a).start()
    _gather(1, rows_b, sem_b).start()

    def _pair(i, c):
        t0 = 2 * i
        t1 = t0 + 1

        @pl.when(i > 0)
        def _():
            pltpu.make_async_copy(
                out_a, out_hbm.at[pl.ds((base + t0 - 2) * _D, _D)],
                sem_oa).wait()
        _gather(t0, rows_a, sem_a).wait()
        _compute(t0, rows_a, out_a)
        pltpu.make_async_copy(
            out_a, out_hbm.at[pl.ds((base + t0) * _D, _D)], sem_oa).start()

        @pl.when(t0 + 2 < _TPW)
        def _():
            _gather(t0 + 2, rows_a, sem_a).start()

        @pl.when(i > 0)
        def _():
            pltpu.make_async_copy(
                out_b, out_hbm.at[pl.ds((base + t1 - 2) * _D, _D)],
                sem_ob).wait()
        _gather(t1, rows_b, sem_b).wait()
        _compute(t1, rows_b, out_b)
        pltpu.make_async_copy(
            out_b, out_hbm.at[pl.ds((base + t1) * _D, _D)], sem_ob).start()

        @pl.when(t1 + 2 < _TPW)
        def _():
            _gather(t1 + 2, rows_b, sem_b).start()

        return c

    lax.fori_loop(0, _TPW // 2, _pair, 0)
    pltpu.make_async_copy(
        out_a, out_hbm.at[pl.ds((base + _TPW - 2) * _D, _D)], sem_oa).wait()
    pltpu.make_async_copy(
        out_b, out_hbm.at[pl.ds((base + _TPW - 1) * _D, _D)], sem_ob).wait()


@jax.jit
def _moe(xflat, param_pool, idx_flat, wts_flat):
    mesh = plsc.VectorSubcoreMesh(core_axis_name="c", subcore_axis_name="s")
    return pl.kernel(
        _moe_tile,
        out_type=jax.ShapeDtypeStruct((_T * _D,), jnp.float32),
        mesh=mesh,
        compiler_params=pltpu.CompilerParams(needs_layout_passes=False),
        scratch_types=[
            pltpu.VMEM((_TPW * _D,), jnp.float32),
            pltpu.VMEM((_TPW * _K,), jnp.int32),
            pltpu.VMEM((_TPW * _K,), jnp.float32),
            pltpu.VMEM((_K, _D), jnp.float32),
            pltpu.VMEM((_K, _D), jnp.float32),
            pltpu.VMEM((_D,), jnp.float32),
            pltpu.VMEM((_D,), jnp.float32),
            pltpu.SemaphoreType.DMA,
            pltpu.SemaphoreType.DMA,
            pltpu.SemaphoreType.DMA,
            pltpu.SemaphoreType.DMA,
        ],
    )(xflat, param_pool, idx_flat, wts_flat)


def kernel(x, param_pool, w_router):
    xf = x.reshape(_T, _D)
    top_idx, weights, psum, cnt = _router(xf, w_router)
    out = _moe(xf.reshape(-1), param_pool,
               top_idx.reshape(-1), weights.reshape(-1))
    counts = cnt.reshape(_S)
    p_mean = psum.reshape(_S) / jnp.float32(_T)
    f = counts / jnp.float32(_T * _K)
    aux_loss = jnp.float32(_S) * jnp.sum(f * p_mean)
    active_count = jnp.sum(counts > 0.0)
    return out.reshape(x.shape), aux_loss, active_count


# speedup vs baseline: 1.4008x; 1.4008x over previous
import functools

import jax
import jax.numpy as jnp
from jax import lax
from jax.experimental import pallas as pl
from jax.experimental.pallas import tpu as pltpu
from jax.experimental.pallas import tpu_sc as plsc

_D = 1024
_S = 16384
_K = 16
_MINK = 4
_T = 2048
_BT = 128
_NSTEPS = _T // _BT
_NW = 32
_TPW = _T // _NW
_LC = 16


def _logits_body(x_ref, w_ref, out_ref):
    out_ref[...] = jnp.dot(x_ref[...], w_ref[...],
                           preferred_element_type=jnp.float32)


_BTM = 512
_BSM = 2048


@jax.jit
def _logits(xf, w_router):
    return pl.pallas_call(
        _logits_body,
        grid=(_T // _BTM, _S // _BSM),
        in_specs=[
            pl.BlockSpec((_BTM, _D), lambda i, j: (i, 0)),
            pl.BlockSpec((_D, _BSM), lambda i, j: (0, j)),
        ],
        out_specs=pl.BlockSpec((_BTM, _BSM), lambda i, j: (i, j)),
        out_shape=jax.ShapeDtypeStruct((_T, _S), jnp.float32),
        compiler_params=pltpu.CompilerParams(
            dimension_semantics=("parallel", "parallel")),
    )(xf, w_router)


def _router_body(l_ref, idx_ref, wts_ref, psum_ref, cnt_ref):
    step = pl.program_id(0)

    @pl.when(step == 0)
    def _():
        psum_ref[...] = jnp.zeros_like(psum_ref)
        cnt_ref[...] = jnp.zeros_like(cnt_ref)

    logits = l_ref[...]

    m = jnp.max(logits, axis=1, keepdims=True)
    e = jnp.exp(logits - m)
    z = jnp.sum(e, axis=1, keepdims=True)
    psum_ref[...] += jnp.sum(e / z, axis=0, keepdims=True)

    iota_s = lax.broadcasted_iota(jnp.int32, logits.shape, 1)
    row = logits
    vals = []
    idxs = []
    neg = jnp.float32(-jnp.inf)
    cnt = jnp.zeros((1, _S), jnp.float32)
    for _ in range(_K):
        mk = jnp.max(row, axis=1, keepdims=True)
        ak = jnp.min(jnp.where(row == mk, iota_s, _S), axis=1, keepdims=True)
        vals.append(mk)
        idxs.append(ak)
        hit = iota_s == ak
        cnt = cnt + jnp.sum(hit.astype(jnp.float32), axis=0, keepdims=True)
        row = jnp.where(hit, neg, row)
    cnt_ref[...] += cnt
    tv = jnp.concatenate(vals, axis=1)
    ti = jnp.concatenate(idxs, axis=1)

    e2 = jnp.exp(tv - tv[:, :1])
    w = e2 / jnp.sum(e2, axis=1, keepdims=True)
    ranks = lax.broadcasted_iota(jnp.int32, w.shape, 1)
    keep = (ranks < _MINK) | (w > (1.0 / _K))
    w = w * keep.astype(jnp.float32)
    w = w / (jnp.sum(w, axis=1, keepdims=True) + 1e-9)

    idx_ref[...] = ti
    wts_ref[...] = w


@jax.jit
def _router(logits):
    return pl.pallas_call(
        _router_body,
        grid=(_NSTEPS,),
        in_specs=[
            pl.BlockSpec((_BT, _S), lambda i: (i, 0)),
        ],
        out_specs=[
            pl.BlockSpec((_BT, _K), lambda i: (i, 0)),
            pl.BlockSpec((_BT, _K), lambda i: (i, 0)),
            pl.BlockSpec((1, _S), lambda i: (0, 0)),
            pl.BlockSpec((1, _S), lambda i: (0, 0)),
        ],
        out_shape=[
            jax.ShapeDtypeStruct((_T, _K), jnp.int32),
            jax.ShapeDtypeStruct((_T, _K), jnp.float32),
            jax.ShapeDtypeStruct((1, _S), jnp.float32),
            jax.ShapeDtypeStruct((1, _S), jnp.float32),
        ],
    )(logits)


def _moe_tile(x_hbm, pp_hbm, idx_hbm, wts_hbm, out_hbm,
              x_v, idx_v, w_v, rows_a, rows_b, out_a, out_b,
              sem_a, sem_b, sem_oa, sem_ob):
    wid = lax.axis_index("s") * 2 + lax.axis_index("c")
    base = wid * _TPW
    lanes = lax.iota(jnp.int32, _LC)
    zero16 = jnp.zeros((_LC,), jnp.float32)

    pltpu.sync_copy(x_hbm.at[pl.ds(base * _D, _TPW * _D)], x_v)
    pltpu.sync_copy(idx_hbm.at[pl.ds(base * _K, _TPW * _K)], idx_v)
    pltpu.sync_copy(wts_hbm.at[pl.ds(base * _K, _TPW * _K)], w_v)

    def _gather(t, rows, sem):
        return pltpu.make_async_copy(
            pp_hbm.at[idx_v.at[pl.ds(t * _K, _K)]], rows, sem)

    def _compute(t, rows, out_v):
        xoff = t * _D

        def _pbody(dc, accs):
            xv = x_v[pl.ds(xoff + dc * _LC, _LC)]
            return tuple(accs[k] + xv * rows[k, pl.ds(dc * _LC, _LC)]
                         for k in range(_K))

        accs = lax.fori_loop(0, _D // _LC, _pbody,
                             tuple(zero16 for _ in range(_K)))

        wv = w_v[pl.ds(t * _K, _K)]
        wps = []
        for k in range(_K):
            s_k = jnp.sum(accs[k], axis=0)
            w_k = jnp.sum(jnp.where(lanes == k, wv, 0.0), axis=0)
            wps.append(jnp.full((_LC,), s_k * w_k, jnp.float32))

        def _obody(dc, c2):
            oc = wps[0] * rows[0, pl.ds(dc * _LC, _LC)]
            for k in range(1, _K):
                oc = oc + wps[k] * rows[k, pl.ds(dc * _LC, _LC)]
            out_v[pl.ds(dc * _LC, _LC)] = oc
            return c2

        lax.fori_loop(0, _D // _LC, _obody, 0)

    _gather(0, rows_a, sem_a).start()
    _gather(1, rows_b, sem_b).start()

    def _pair(i, c):
        t0 = 2 * i
        t1 = t0 + 1

        @pl.when(i > 0)
        def _():
            pltpu.make_async_copy(
                out_a, out_hbm.at[pl.ds((base + t0 - 2) * _D, _D)],
                sem_oa).wait()
        _gather(t0, rows_a, sem_a).wait()
        _compute(t0, rows_a, out_a)
        pltpu.make_async_copy(
            out_a, out_hbm.at[pl.ds((base + t0) * _D, _D)], sem_oa).start()

        @pl.when(t0 + 2 < _TPW)
        def _():
            _gather(t0 + 2, rows_a, sem_a).start()

        @pl.when(i > 0)
        def _():
            pltpu.make_async_copy(
                out_b, out_hbm.at[pl.ds((base + t1 - 2) * _D, _D)],
                sem_ob).wait()
        _gather(t1, rows_b, sem_b).wait()
        _compute(t1, rows_b, out_b)
        pltpu.make_async_copy(
            out_b, out_hbm.at[pl.ds((base + t1) * _D, _D)], sem_ob).start()

        @pl.when(t1 + 2 < _TPW)
        def _():
            _gather(t1 + 2, rows_b, sem_b).start()

        return c

    lax.fori_loop(0, _TPW // 2, _pair, 0)
    pltpu.make_async_copy(
        out_a, out_hbm.at[pl.ds((base + _TPW - 2) * _D, _D)], sem_oa).wait()
    pltpu.make_async_copy(
        out_b, out_hbm.at[pl.ds((base + _TPW - 1) * _D, _D)], sem_ob).wait()


@jax.jit
def _moe(xflat, param_pool, idx_flat, wts_flat):
    mesh = plsc.VectorSubcoreMesh(core_axis_name="c", subcore_axis_name="s")
    return pl.kernel(
        _moe_tile,
        out_type=jax.ShapeDtypeStruct((_T * _D,), jnp.float32),
        mesh=mesh,
        compiler_params=pltpu.CompilerParams(needs_layout_passes=False),
        scratch_types=[
            pltpu.VMEM((_TPW * _D,), jnp.float32),
            pltpu.VMEM((_TPW * _K,), jnp.int32),
            pltpu.VMEM((_TPW * _K,), jnp.float32),
            pltpu.VMEM((_K, _D), jnp.float32),
            pltpu.VMEM((_K, _D), jnp.float32),
            pltpu.VMEM((_D,), jnp.float32),
            pltpu.VMEM((_D,), jnp.float32),
            pltpu.SemaphoreType.DMA,
            pltpu.SemaphoreType.DMA,
            pltpu.SemaphoreType.DMA,
            pltpu.SemaphoreType.DMA,
        ],
    )(xflat, param_pool, idx_flat, wts_flat)


def kernel(x, param_pool, w_router):
    xf = x.reshape(_T, _D)
    logits = _logits(xf, w_router)
    top_idx, weights, psum, cnt = _router(logits)
    out = _moe(xf.reshape(-1), param_pool,
               top_idx.reshape(-1), weights.reshape(-1))
    counts = cnt.reshape(_S)
    p_mean = psum.reshape(_S) / jnp.float32(_T)
    f = counts / jnp.float32(_T * _K)
    aux_loss = jnp.float32(_S) * jnp.sum(f * p_mean)
    active_count = jnp.sum(counts > 0.0)
    return out.reshape(x.shape), aux_loss, active_count
